# baseline (speedup 1.0000x reference)
import functools

import jax
import jax.numpy as jnp
from jax import lax
from jax.experimental import pallas as pl
from jax.experimental.pallas import tpu as pltpu
from jax.experimental.pallas import tpu_sc as plsc

NUM_CLASSES = 1000000
EMBED_DIM = 64
BATCH = 16384

_NUM_CORES = 2
_NUM_SUBCORES = 16
_NUM_TILES = _NUM_CORES * _NUM_SUBCORES
_B_PER_TILE = BATCH // _NUM_TILES


def _sc_gather(table, idx):
    mesh = plsc.VectorSubcoreMesh(core_axis_name="c", subcore_axis_name="s")

    @functools.partial(
        pl.kernel,
        mesh=mesh,
        out_type=jax.ShapeDtypeStruct((BATCH, EMBED_DIM), table.dtype),
        scratch_types=[
            pltpu.VMEM((_B_PER_TILE,), jnp.int32),
            pltpu.SemaphoreType.DMA,
        ],
    )
    def gather_kernel(table_hbm, idx_hbm, out_hbm, idx_v, sem):
        wid = lax.axis_index("s") * _NUM_CORES + lax.axis_index("c")
        base = wid * _B_PER_TILE
        pltpu.sync_copy(idx_hbm.at[pl.ds(base, _B_PER_TILE)], idx_v)

        @pl.loop(0, _B_PER_TILE, step=16)
        def _fire(j0):
            v = idx_v[pl.ds(j0, 16)]
            for k in range(16):
                pltpu.async_copy(
                    table_hbm.at[v[k]], out_hbm.at[base + j0 + k], sem
                )

        @pl.loop(0, _B_PER_TILE)
        def _drain(j):
            pltpu.make_async_copy(
                table_hbm.at[0], out_hbm.at[base + j], sem
            ).wait()

    return gather_kernel(table, idx)


def _mlp_block_kernel(e_ref, w1_ref, b1_ref, w2_ref, b2_ref, o_ref):
    h = jnp.dot(e_ref[...], w1_ref[...], preferred_element_type=jnp.float32)
    h = h + b1_ref[...]
    h = h * jax.nn.sigmoid(h)
    h = jnp.dot(h, w2_ref[...], preferred_element_type=jnp.float32)
    h = h + b2_ref[...]
    o_ref[...] = h * jax.nn.sigmoid(h)


def _tc_mlp(emb, W1t, b1, W2t, b2):
    blk = 2048
    grid = (BATCH // blk,)
    return pl.pallas_call(
        _mlp_block_kernel,
        grid=grid,
        in_specs=[
            pl.BlockSpec((blk, EMBED_DIM), lambda i: (i, 0)),
            pl.BlockSpec((EMBED_DIM, EMBED_DIM), lambda i: (0, 0)),
            pl.BlockSpec((1, EMBED_DIM), lambda i: (0, 0)),
            pl.BlockSpec((EMBED_DIM, EMBED_DIM), lambda i: (0, 0)),
            pl.BlockSpec((1, EMBED_DIM), lambda i: (0, 0)),
        ],
        out_specs=pl.BlockSpec((blk, EMBED_DIM), lambda i: (i, 0)),
        out_shape=jax.ShapeDtypeStruct((BATCH, EMBED_DIM), jnp.float32),
    )(emb, W1t, b1.reshape(1, EMBED_DIM), W2t, b2.reshape(1, EMBED_DIM))


def kernel(x, table, W1, b1, W2, b2):
    idx = x.astype(jnp.int32)
    emb = _sc_gather(table, idx)
    return _tc_mlp(emb, W1.T, b1, W2.T, b2)


# speedup vs baseline: 1.6478x; 1.6478x over previous
import functools

import jax
import jax.numpy as jnp
from jax import lax
from jax.experimental import pallas as pl
from jax.experimental.pallas import tpu as pltpu
from jax.experimental.pallas import tpu_sc as plsc

NUM_CLASSES = 1000000
EMBED_DIM = 64
BATCH = 16384

_NUM_CORES = 2
_NUM_SUBCORES = 16
_NUM_TILES = _NUM_CORES * _NUM_SUBCORES
_B_PER_TILE = BATCH // _NUM_TILES


_CHUNK = 64
_N_CHUNKS = _B_PER_TILE // _CHUNK


def _sc_gather(table, idx):
    mesh = plsc.VectorSubcoreMesh(core_axis_name="c", subcore_axis_name="s")

    @functools.partial(
        pl.kernel,
        mesh=mesh,
        out_type=jax.ShapeDtypeStruct((BATCH, EMBED_DIM), table.dtype),
        scratch_types=[
            pltpu.VMEM((_B_PER_TILE,), jnp.int32),
            pltpu.VMEM((_B_PER_TILE, EMBED_DIM), jnp.float32),
            pltpu.SemaphoreType.DMA,
        ],
    )
    def gather_kernel(table_hbm, idx_hbm, out_hbm, idx_v, rows_v, sem):
        wid = lax.axis_index("s") * _NUM_CORES + lax.axis_index("c")
        base = wid * _B_PER_TILE
        pltpu.sync_copy(idx_hbm.at[pl.ds(base, _B_PER_TILE)], idx_v)

        @pl.loop(0, _B_PER_TILE, step=16)
        def _fire(j0):
            v = idx_v[pl.ds(j0, 16)]
            for k in range(16):
                pltpu.async_copy(
                    table_hbm.at[v[k]], rows_v.at[j0 + k], sem
                )

        @pl.loop(0, _B_PER_TILE)
        def _drain(j):
            pltpu.make_async_copy(
                table_hbm.at[0], rows_v.at[j], sem
            ).wait()

        pltpu.sync_copy(rows_v, out_hbm.at[pl.ds(base, _B_PER_TILE)])

    return gather_kernel(table, idx)


def _mlp_block_kernel(e_ref, w1_ref, b1_ref, w2_ref, b2_ref, o_ref):
    h = jnp.dot(e_ref[...], w1_ref[...], preferred_element_type=jnp.float32)
    h = h + b1_ref[...]
    h = h * jax.nn.sigmoid(h)
    h = jnp.dot(h, w2_ref[...], preferred_element_type=jnp.float32)
    h = h + b2_ref[...]
    o_ref[...] = h * jax.nn.sigmoid(h)


def _tc_mlp(emb, W1t, b1, W2t, b2):
    blk = 2048
    grid = (BATCH // blk,)
    return pl.pallas_call(
        _mlp_block_kernel,
        grid=grid,
        in_specs=[
            pl.BlockSpec((blk, EMBED_DIM), lambda i: (i, 0)),
            pl.BlockSpec((EMBED_DIM, EMBED_DIM), lambda i: (0, 0)),
            pl.BlockSpec((1, EMBED_DIM), lambda i: (0, 0)),
            pl.BlockSpec((EMBED_DIM, EMBED_DIM), lambda i: (0, 0)),
            pl.BlockSpec((1, EMBED_DIM), lambda i: (0, 0)),
        ],
        out_specs=pl.BlockSpec((blk, EMBED_DIM), lambda i: (i, 0)),
        out_shape=jax.ShapeDtypeStruct((BATCH, EMBED_DIM), jnp.float32),
    )(emb, W1t, b1.reshape(1, EMBED_DIM), W2t, b2.reshape(1, EMBED_DIM))


def kernel(x, table, W1, b1, W2, b2):
    idx = x.astype(jnp.int32)
    emb = _sc_gather(table, idx)
    return _tc_mlp(emb, W1.T, b1, W2.T, b2)
